# baseline (speedup 1.0000x reference)
import jax
import jax.numpy as jnp
from jax.experimental import pallas as pl

_BLOCK_S = 512


def _copy_kernel(emb_ref, out_ref):
    out_ref[...] = emb_ref[...][None]


def kernel(x, pos_emb):
    batch, seq_len = x.shape
    max_len, d_model = pos_emb.shape
    grid = (seq_len // _BLOCK_S, batch)
    return pl.pallas_call(
        _copy_kernel,
        grid=grid,
        in_specs=[pl.BlockSpec((_BLOCK_S, d_model), lambda i, j: (i, 0))],
        out_specs=pl.BlockSpec((1, _BLOCK_S, d_model), lambda i, j: (j, i, 0)),
        out_shape=jax.ShapeDtypeStruct((batch, seq_len, d_model), pos_emb.dtype),
    )(pos_emb)


# speedup vs baseline: 1.5154x; 1.5154x over previous
import jax
import jax.numpy as jnp
from jax.experimental import pallas as pl

_BLOCK_S = 1024


def _bcast_copy_kernel(emb_ref, out_ref):
    out_ref[...] = jnp.broadcast_to(emb_ref[...][None], out_ref.shape)


def kernel(x, pos_emb):
    batch, seq_len = x.shape
    max_len, d_model = pos_emb.shape
    grid = (seq_len // _BLOCK_S,)
    return pl.pallas_call(
        _bcast_copy_kernel,
        grid=grid,
        in_specs=[pl.BlockSpec((_BLOCK_S, d_model), lambda i: (i, 0))],
        out_specs=pl.BlockSpec((batch, _BLOCK_S, d_model), lambda i: (0, i, 0)),
        out_shape=jax.ShapeDtypeStruct((batch, seq_len, d_model), pos_emb.dtype),
    )(pos_emb)
